# Initial kernel scaffold; baseline (speedup 1.0000x reference)
import jax
import jax.numpy as jnp
from jax.experimental import pallas as pl


def kernel(node_embeddings, W_in, b_in, ln_in_g, ln_in_b, temporal_query, general_query, Wk, bk, Wv, bv, Wo_pool, bo_pool, ln_pool_g, ln_pool_b, W_attn, b_attn, W_attn_out, b_attn_out, ln1_g, ln1_b, ln2_g, ln2_b, W_ff1, b_ff1, W_ff2, b_ff2, W_r1, b_r1, W_r2, b_r2, commit_indices, is_temporal_node):
    raise NotImplementedError("write your pallas kernel here")


# speedup vs baseline: 9.4700x; 9.4700x over previous
import functools

import jax
import jax.numpy as jnp
from jax import lax
from jax.experimental import pallas as pl
from jax.experimental.pallas import tpu as pltpu
from jax.experimental.pallas import tpu_sc as plsc

N = 50000
D_IN = 512
D = 256
H = 4
HD = 64
C = 2048

BLK = 1024
NP = 50176
NBLK = NP // BLK
UW = 384

NWORK = 32
ROWS_PER_W = NP // NWORK
CHUNK = 112
NCHUNK = ROWS_PER_W // CHUNK

_SCALE = HD ** -0.5
_NEG = -1e30


def _gelu(x):
    return 0.5 * x * (1.0 + lax.erf(x * (2.0 ** -0.5)))


def _ln(x, g, b, eps=1e-5):
    m = jnp.mean(x, axis=-1, keepdims=True)
    v = jnp.mean((x - m) ** 2, axis=-1, keepdims=True)
    return (x - m) * lax.rsqrt(v + eps) * g + b


def _a_body(x_ref, mt_ref, win_ref, bin_ref, lg_ref, lb_ref, wk_ref, bk_ref,
            wv_ref, bv_ref, tq_ref, gq_ref, vlog_ref, max_ref):
    i = pl.program_id(0)
    h = _ln(x_ref[...] @ win_ref[...] + bin_ref[...], lg_ref[...], lb_ref[...])
    h = _gelu(h)
    k = h @ wk_ref[...] + bk_ref[...]
    v = h @ wv_ref[...] + bv_ref[...]
    mtv = mt_ref[...]
    q = gq_ref[...] + mtv * (tq_ref[...] - gq_ref[...])
    kq = k * q
    cols = [jnp.sum(kq[:, j * HD:(j + 1) * HD], axis=1, keepdims=True)
            for j in range(H)]
    logits = jnp.concatenate(cols, axis=1) * _SCALE
    row = i * BLK + lax.broadcasted_iota(jnp.int32, (BLK, H), 0)
    vf = jnp.minimum(1.0, jnp.maximum(0.0, (N - row).astype(jnp.float32)))
    logits = logits * vf + (1.0 - vf) * _NEG
    vlog_ref[...] = jnp.concatenate(
        [v, logits, jnp.zeros((BLK, UW - D - H), jnp.float32)], axis=1)
    bm = jnp.max(logits)

    @pl.when(i == 0)
    def _():
        max_ref[0, 0] = bm

    @pl.when(i > 0)
    def _():
        max_ref[0, 0] = jnp.maximum(max_ref[0, 0], bm)


def _stage_a(xp, mt, w_in, b_in, lg, lb, wk, bk, wv, bv, tq, gq):
    full = lambda s: pl.BlockSpec(s, lambda i: (0, 0))
    return pl.pallas_call(
        _a_body,
        grid=(NBLK,),
        in_specs=[
            pl.BlockSpec((BLK, D_IN), lambda i: (i, 0)),
            pl.BlockSpec((BLK, 1), lambda i: (i, 0)),
            full((D_IN, D)), full((1, D)), full((1, D)), full((1, D)),
            full((D, D)), full((1, D)), full((D, D)), full((1, D)),
            full((1, D)), full((1, D)),
        ],
        out_specs=[
            pl.BlockSpec((BLK, UW), lambda i: (i, 0)),
            pl.BlockSpec(memory_space=pltpu.SMEM),
        ],
        out_shape=[
            jax.ShapeDtypeStruct((NP, UW), jnp.float32),
            jax.ShapeDtypeStruct((1, 1), jnp.float32),
        ],
    )(xp, mt, w_in, b_in, lg, lb, wk, bk, wv, bv, tq, gq)


def _a2_body(vl_ref, m_ref, u_ref):
    vl = vl_ref[...]
    e = jnp.exp(vl[:, D:D + H] - m_ref[0, 0])
    cols = [vl[:, j * HD:(j + 1) * HD] * e[:, j:j + 1] for j in range(H)]
    cols.append(e)
    cols.append(jnp.zeros((BLK, UW - D - H), jnp.float32))
    u_ref[...] = jnp.concatenate(cols, axis=1)


def _stage_a2(vlog, mx):
    return pl.pallas_call(
        _a2_body,
        grid=(NBLK,),
        in_specs=[
            pl.BlockSpec((BLK, UW), lambda i: (i, 0)),
            pl.BlockSpec(memory_space=pltpu.SMEM),
        ],
        out_specs=pl.BlockSpec((BLK, UW), lambda i: (i, 0)),
        out_shape=jax.ShapeDtypeStruct((NP, UW), jnp.float32),
    )(vlog, mx)


def _b_body(vl_ref, ci_ref, m_ref, out_ref):
    i = pl.program_id(0)

    @pl.when(i == 0)
    def _():
        out_ref[...] = jnp.zeros((C, UW), jnp.float32)

    vl = vl_ref[...]
    e = jnp.exp(vl[:, D:D + H] - m_ref[0, 0])
    cols = [vl[:, j * HD:(j + 1) * HD] * e[:, j:j + 1] for j in range(H)]
    cols.append(e)
    cols.append(jnp.zeros((BLK, UW - D - H), jnp.float32))
    ub = jnp.concatenate(cols, axis=1).astype(jnp.bfloat16)
    cf = ci_ref[...].astype(jnp.float32)
    io = lax.broadcasted_iota(jnp.int32, (BLK, C), 1).astype(jnp.float32)
    oh = jnp.maximum(0.0, 1.0 - jnp.abs(io - cf)).astype(jnp.bfloat16)
    out_ref[...] += lax.dot_general(
        oh, ub, (((0,), (0,)), ((), ())),
        preferred_element_type=jnp.float32)


def _stage_b(vlog, cip2, mx):
    return pl.pallas_call(
        _b_body,
        grid=(NBLK,),
        in_specs=[
            pl.BlockSpec((BLK, UW), lambda i: (i, 0)),
            pl.BlockSpec((BLK, 1), lambda i: (i, 0)),
            pl.BlockSpec(memory_space=pltpu.SMEM),
        ],
        out_specs=pl.BlockSpec((C, UW), lambda i: (0, 0)),
        out_shape=jax.ShapeDtypeStruct((C, UW), jnp.float32),
    )(vlog, cip2, mx)


def _c1_body(p_ref, wo_ref, bo_ref, lpg_ref, lpb_ref, wa_ref, ba_ref,
             x_ref, qkv_ref):
    p = p_ref[...]
    esum = p[:, D:D + H]
    cols = [p[:, j * HD:(j + 1) * HD] / (esum[:, j:j + 1] + 1e-9)
            for j in range(H)]
    pooled = jnp.concatenate(cols, axis=1)
    x = _ln(pooled @ wo_ref[...] + bo_ref[...], lpg_ref[...], lpb_ref[...])
    x_ref[...] = x
    qkv_ref[...] = x @ wa_ref[...] + ba_ref[...]


def _att_body(qt_ref, kt_ref, vt_ref, ot_ref):
    st = lax.dot_general(kt_ref[...], qt_ref[...],
                         (((0,), (0,)), ((), ()))) * _SCALE
    st = st - jnp.max(st, axis=0, keepdims=True)
    e = jnp.exp(st)
    p = e / jnp.sum(e, axis=0, keepdims=True)
    ot_ref[...] = lax.dot_general(vt_ref[...], p, (((1,), (0,)), ((), ())))


def _c3_body(x_ref, ao_ref, wao_ref, bao_ref, l1g_ref, l1b_ref, l2g_ref,
             l2b_ref, wf1_ref, bf1_ref, wf2_ref, bf2_ref, wr1_ref, br1_ref,
             wr2_ref, br2_ref, out_ref):
    ao = ao_ref[...] @ wao_ref[...] + bao_ref[...]
    x = _ln(x_ref[...] + ao, l1g_ref[...], l1b_ref[...])
    ff = _gelu(x @ wf1_ref[...] + bf1_ref[...]) @ wf2_ref[...] + bf2_ref[...]
    x = _ln(x + ff, l2g_ref[...], l2b_ref[...])
    r = _gelu(x @ wr1_ref[...] + br1_ref[...])
    s = jnp.sum(r * wr2_ref[...], axis=1, keepdims=True) + br2_ref[...]
    out_ref[...] = s + jnp.zeros((C, 128), jnp.float32)


def _one_block(body, args, out_shape):
    return pl.pallas_call(
        body,
        in_specs=[pl.BlockSpec(a.shape, lambda: (0,) * a.ndim) for a in args],
        out_specs=jax.tree.map(
            lambda s: pl.BlockSpec(s.shape, lambda: (0,) * len(s.shape)),
            out_shape),
        out_shape=out_shape,
    )(*args)


def kernel(node_embeddings, W_in, b_in, ln_in_g, ln_in_b, temporal_query,
           general_query, Wk, bk, Wv, bv, Wo_pool, bo_pool, ln_pool_g,
           ln_pool_b, W_attn, b_attn, W_attn_out, b_attn_out, ln1_g, ln1_b,
           ln2_g, ln2_b, W_ff1, b_ff1, W_ff2, b_ff2, W_r1, b_r1, W_r2, b_r2,
           commit_indices, is_temporal_node):
    pad = NP - N
    xp = jnp.pad(node_embeddings, ((0, pad), (0, 0)))
    mt = jnp.pad(is_temporal_node.astype(jnp.float32), (0, pad)).reshape(NP, 1)
    cip = jnp.pad(commit_indices.astype(jnp.int32), (0, pad),
                  constant_values=C - 1)
    r1 = lambda a: a.reshape(1, -1)
    vlog, mx = _stage_a(xp, mt, W_in, r1(b_in), r1(ln_in_g), r1(ln_in_b),
                        Wk, r1(bk), Wv, r1(bv), r1(temporal_query),
                        r1(general_query))
    pooled_sums = _stage_b(vlog, cip.reshape(NP, 1), mx)

    x, qkv = _one_block(
        _c1_body,
        (pooled_sums, Wo_pool,
         r1(bo_pool), r1(ln_pool_g), r1(ln_pool_b), W_attn, r1(b_attn)),
        (jax.ShapeDtypeStruct((C, D), jnp.float32),
         jax.ShapeDtypeStruct((C, 3 * D), jnp.float32)))
    aos = []
    for j in range(H):
        aos.append(_one_block(
            _att_body,
            (qkv[:, j * HD:(j + 1) * HD].T,
             qkv[:, D + j * HD:D + (j + 1) * HD].T,
             qkv[:, 2 * D + j * HD:2 * D + (j + 1) * HD].T),
            jax.ShapeDtypeStruct((HD, C), jnp.float32)).T)
    ao = jnp.concatenate(aos, axis=1)
    out = _one_block(
        _c3_body,
        (x, ao, W_attn_out, r1(b_attn_out), r1(ln1_g), r1(ln1_b), r1(ln2_g),
         r1(ln2_b), W_ff1, r1(b_ff1), W_ff2, r1(b_ff2), W_r1, r1(b_r1),
         W_r2.reshape(1, D // 2), r1(b_r2)),
        jax.ShapeDtypeStruct((C, 128), jnp.float32))
    return out[:, 0]
